# Initial kernel scaffold; baseline (speedup 1.0000x reference)
import jax
import jax.numpy as jnp
from jax.experimental import pallas as pl


def kernel(x, edge_index, lin_W, lin_b, c1_Wsrc, c1_Wdst, c1_asrc, c1_adst, c1_b, c2_Wsrc, c2_Wdst, c2_asrc, c2_adst, c2_b):
    raise NotImplementedError("write your pallas kernel here")


# speedup vs baseline: 21.3916x; 21.3916x over previous
import dataclasses
import functools

import jax
import jax.numpy as jnp
from jax import lax
from jax.experimental import pallas as pl
from jax.experimental.pallas import tpu as pltpu
from jax.experimental.pallas import tpu_sc as plsc

N = 10000
E = 320000
D = 128
H = 128
L = 64

NC = 2
NS = 16
NW = NC * NS
EPW = E // NW
K = 80
NCHUNK = EPW // K
RB = 80
NRB = N // RB

F32 = jnp.float32


def _pre1_body(x_ref, wsrc_ref, asrc_ref, wdst_ref, adst_ref, linw_ref,
               linb_ref, h_ref, as_ref, ad_ref, zlin_ref):
    x = x_ref[...]
    h = jnp.dot(x, wsrc_ref[...], preferred_element_type=F32)
    h_ref[...] = h
    as_ref[...] = lax.dot_general(h, asrc_ref[...],
                                  (((1,), (1,)), ((), ())),
                                  preferred_element_type=F32)
    v = lax.dot_general(wdst_ref[...], adst_ref[...],
                        (((1,), (1,)), ((), ())),
                        preferred_element_type=F32)
    ad_ref[...] = jnp.dot(x, v, preferred_element_type=F32)
    zlin_ref[...] = lax.dot_general(x, linw_ref[...],
                                    (((1,), (1,)), ((), ())),
                                    preferred_element_type=F32) + linb_ref[...]


def _tc_pre1(x, wsrc, asrc, wdst, adst, linw, linb):
    return pl.pallas_call(
        _pre1_body,
        out_shape=[
            jax.ShapeDtypeStruct((N, H), F32),
            jax.ShapeDtypeStruct((N, 1), F32),
            jax.ShapeDtypeStruct((N, 1), F32),
            jax.ShapeDtypeStruct((N, D), F32),
        ],
    )(x, wsrc, asrc, wdst, adst, linw, linb)


def _rcp_body(dp_ref, out_ref):
    out_ref[...] = 1.0 / (jnp.sum(dp_ref[...], axis=0, keepdims=True) + 1e-16)


def _tc_rcp(den_parts):
    return pl.pallas_call(
        _rcp_body,
        out_shape=jax.ShapeDtypeStruct((1, N), F32),
    )(den_parts)


def _mid_body(acc_ref, zlin_ref, c1b_ref, wsrc_ref, asrc_ref, wdst_ref,
              adst_ref, h_ref, as_ref, ad_ref):
    g = acc_ref[0] + acc_ref[1] + c1b_ref[...]
    z = jnp.maximum(g + zlin_ref[...], 0.0)
    h = jnp.dot(z, wsrc_ref[...], preferred_element_type=F32)
    h_ref[...] = jnp.concatenate([h, jnp.zeros((N, D - L), F32)], axis=1)
    as_ref[...] = lax.dot_general(h, asrc_ref[...],
                                  (((1,), (1,)), ((), ())),
                                  preferred_element_type=F32)
    v = lax.dot_general(wdst_ref[...], adst_ref[...],
                        (((1,), (1,)), ((), ())),
                        preferred_element_type=F32)
    ad_ref[...] = jnp.dot(z, v, preferred_element_type=F32)


def _tc_mid(acc1, zlin, c1b, wsrc2, asrc2, wdst2, adst2):
    return pl.pallas_call(
        _mid_body,
        out_shape=[
            jax.ShapeDtypeStruct((N, D), F32),
            jax.ShapeDtypeStruct((N, 1), F32),
            jax.ShapeDtypeStruct((N, 1), F32),
        ],
    )(acc1, zlin, c1b, wsrc2, asrc2, wdst2, adst2)


def _post_body(acc_ref, c2b_ref, out_ref):
    s = acc_ref[0] + acc_ref[1]
    out_ref[...] = s[:, :L] + c2b_ref[...]


def _tc_post(acc2, c2b):
    return pl.pallas_call(
        _post_body,
        out_shape=jax.ShapeDtypeStruct((N, L), F32),
    )(acc2, c2b)


_MESH = plsc.VectorSubcoreMesh(core_axis_name="c", subcore_axis_name="s")

_SC_CP = pltpu.CompilerParams()
if "needs_layout_passes" in pltpu.CompilerParams.__dataclass_fields__:
    _SC_CP = dataclasses.replace(_SC_CP, needs_layout_passes=False)


def _lrelu_exp(raw):
    return jnp.exp(jnp.where(raw >= 0.0, raw, raw * F32(0.2)))


@functools.partial(
    pl.kernel,
    out_type=jax.ShapeDtypeStruct((NW, N), F32),
    mesh=_MESH,
    scratch_types=[
        pltpu.VMEM((N,), F32),
        pltpu.VMEM((N,), F32),
        pltpu.VMEM((N,), F32),
        pltpu.VMEM((EPW,), jnp.int32),
        pltpu.VMEM((EPW,), jnp.int32),
    ],
    compiler_params=_SC_CP,
)
def _sc_denom(src_hbm, dst_hbm, asrc_hbm, adst_hbm, out_hbm,
              asrc_v, adst_v, den_v, src_v, dst_v):
    cid = lax.axis_index("c")
    sid = lax.axis_index("s")
    wid = cid * NS + sid

    pltpu.sync_copy(asrc_hbm, asrc_v)
    pltpu.sync_copy(adst_hbm, adst_v)
    pltpu.sync_copy(src_hbm.at[pl.ds(wid * EPW, EPW)], src_v)
    pltpu.sync_copy(dst_hbm.at[pl.ds(wid * EPW, EPW)], dst_v)

    z16 = jnp.zeros((16,), F32)

    @pl.loop(0, N, step=16)
    def _(i):
        den_v[pl.ds(i, 16)] = z16

    @pl.loop(0, EPW, step=16)
    def _(i):
        s16 = src_v[pl.ds(i, 16)]
        d16 = dst_v[pl.ds(i, 16)]
        raw = plsc.load_gather(asrc_v, [s16]) + plsc.load_gather(adst_v, [d16])
        plsc.addupdate_scatter(den_v, [d16], _lrelu_exp(raw))

    pltpu.sync_copy(den_v, out_hbm.at[wid])


@functools.partial(
    pl.kernel,
    out_type=jax.ShapeDtypeStruct((NC, N, D), F32),
    mesh=_MESH,
    scratch_types=[
        pltpu.VMEM((N,), F32),
        pltpu.VMEM((N,), F32),
        pltpu.VMEM((N,), F32),
        pltpu.VMEM((1, K), jnp.int32),
        pltpu.VMEM((1, K), jnp.int32),
        pltpu.VMEM((K, D), F32),
        pltpu.VMEM((K,), F32),
        pltpu.VMEM_SHARED((N, D), F32),
        pltpu.SemaphoreType.DMA,
    ],
    compiler_params=_SC_CP,
)
def _sc_messages(h_hbm, src_hbm, dst_hbm, asrc_hbm, adst_hbm, rden_hbm,
                 out_hbm, asrc_v, adst_v, rden_v, src_v, dst_v, rows_v,
                 w_v, acc_sh, sem):
    cid = lax.axis_index("c")
    sid = lax.axis_index("s")
    wid = cid * NS + sid

    pltpu.sync_copy(asrc_hbm, asrc_v)
    pltpu.sync_copy(adst_hbm, adst_v)
    pltpu.sync_copy(rden_hbm, rden_v)

    z16 = jnp.zeros((16,), F32)

    @pl.loop(0, RB)
    def _(r):
        @pl.loop(0, D, step=16)
        def _(j):
            rows_v[r, pl.ds(j, 16)] = z16

    @pl.loop(sid, NRB, step=NS)
    def _(m):
        pltpu.sync_copy(rows_v, acc_sh.at[pl.ds(m * RB, RB)])

    plsc.subcore_barrier()

    ebase = wid * EPW

    @pl.loop(0, NCHUNK)
    def _(c):
        b = ebase + c * K
        pltpu.sync_copy(src_hbm.at[pl.ds(b, K)], src_v.at[0])
        pltpu.sync_copy(dst_hbm.at[pl.ds(b, K)], dst_v.at[0])
        pltpu.async_copy(h_hbm.at[src_v.at[0]], rows_v, sem).wait()

        @pl.loop(0, K, step=16)
        def _(i):
            s16 = src_v[0, pl.ds(i, 16)]
            d16 = dst_v[0, pl.ds(i, 16)]
            raw = (plsc.load_gather(asrc_v, [s16])
                   + plsc.load_gather(adst_v, [d16]))
            w_v[pl.ds(i, 16)] = (_lrelu_exp(raw)
                                 * plsc.load_gather(rden_v, [d16]))

        @pl.loop(0, K)
        def _(r):
            ws = plsc.load_gather(w_v, [jnp.full((16,), r, jnp.int32)])
            for j in range(0, D, 16):
                rows_v[r, pl.ds(j, 16)] = rows_v[r, pl.ds(j, 16)] * ws

        pltpu.sync_copy(rows_v, acc_sh.at[dst_v.at[0]], add=True)

    plsc.subcore_barrier()

    @pl.loop(sid, NRB, step=NS)
    def _(m):
        pltpu.sync_copy(acc_sh.at[pl.ds(m * RB, RB)],
                        out_hbm.at[cid, pl.ds(m * RB, RB)])


def kernel(x, edge_index, lin_W, lin_b, c1_Wsrc, c1_Wdst, c1_asrc, c1_adst,
           c1_b, c2_Wsrc, c2_Wdst, c2_asrc, c2_adst, c2_b):
    src = edge_index[0]
    dst = edge_index[1]

    h1, as1, ad1, zlin = _tc_pre1(
        x, c1_Wsrc, c1_asrc.reshape(1, H), c1_Wdst, c1_adst.reshape(1, H),
        lin_W, lin_b.reshape(1, H))
    dp1 = _sc_denom(src, dst, as1.reshape(N), ad1.reshape(N))
    rden1 = _tc_rcp(dp1).reshape(N)
    acc1 = _sc_messages(h1, src, dst, as1.reshape(N), ad1.reshape(N), rden1)

    h2, as2, ad2 = _tc_mid(
        acc1, zlin, c1_b.reshape(1, H), c2_Wsrc, c2_asrc.reshape(1, L),
        c2_Wdst, c2_adst.reshape(1, L))
    dp2 = _sc_denom(src, dst, as2.reshape(N), ad2.reshape(N))
    rden2 = _tc_rcp(dp2).reshape(N)
    acc2 = _sc_messages(h2, src, dst, as2.reshape(N), ad2.reshape(N), rden2)

    return _tc_post(acc2, c2_b.reshape(1, L))


# speedup vs baseline: 29.0701x; 1.3589x over previous
import dataclasses
import functools

import jax
import jax.numpy as jnp
from jax import lax
from jax.experimental import pallas as pl
from jax.experimental.pallas import tpu as pltpu
from jax.experimental.pallas import tpu_sc as plsc

N = 10000
E = 320000
D = 128
H = 128
L = 64

NC = 2
NS = 16
NW = NC * NS
EPW = E // NW
K = 80
NCHUNK = EPW // K
RB = 80
NRB = N // RB

F32 = jnp.float32


def _pre1_body(x_ref, wsrc_ref, asrc_ref, wdst_ref, adst_ref, linw_ref,
               linb_ref, h_ref, as_ref, ad_ref, zlin_ref):
    x = x_ref[...]
    h = jnp.dot(x, wsrc_ref[...], preferred_element_type=F32)
    h_ref[...] = h
    as_ref[...] = lax.dot_general(h, asrc_ref[...],
                                  (((1,), (1,)), ((), ())),
                                  preferred_element_type=F32)
    v = lax.dot_general(wdst_ref[...], adst_ref[...],
                        (((1,), (1,)), ((), ())),
                        preferred_element_type=F32)
    ad_ref[...] = jnp.dot(x, v, preferred_element_type=F32)
    zlin_ref[...] = lax.dot_general(x, linw_ref[...],
                                    (((1,), (1,)), ((), ())),
                                    preferred_element_type=F32) + linb_ref[...]


def _tc_pre1(x, wsrc, asrc, wdst, adst, linw, linb):
    return pl.pallas_call(
        _pre1_body,
        out_shape=[
            jax.ShapeDtypeStruct((N, H), F32),
            jax.ShapeDtypeStruct((N, 1), F32),
            jax.ShapeDtypeStruct((N, 1), F32),
            jax.ShapeDtypeStruct((N, D), F32),
        ],
    )(x, wsrc, asrc, wdst, adst, linw, linb)


def _rcp_body(dp_ref, out_ref):
    out_ref[...] = 1.0 / (jnp.sum(dp_ref[...], axis=0, keepdims=True) + 1e-16)


def _tc_rcp(den_parts):
    return pl.pallas_call(
        _rcp_body,
        out_shape=jax.ShapeDtypeStruct((1, N), F32),
    )(den_parts)


def _mid_body(acc_ref, rden_ref, zlin_ref, c1b_ref, wsrc_ref, asrc_ref,
              wdst_ref, adst_ref, h_ref, as_ref, ad_ref):
    g = (acc_ref[0] + acc_ref[1]) * rden_ref[...] + c1b_ref[...]
    z = jnp.maximum(g + zlin_ref[...], 0.0)
    h = jnp.dot(z, wsrc_ref[...], preferred_element_type=F32)
    h_ref[...] = jnp.concatenate([h, jnp.zeros((N, D - L), F32)], axis=1)
    as_ref[...] = lax.dot_general(h, asrc_ref[...],
                                  (((1,), (1,)), ((), ())),
                                  preferred_element_type=F32)
    v = lax.dot_general(wdst_ref[...], adst_ref[...],
                        (((1,), (1,)), ((), ())),
                        preferred_element_type=F32)
    ad_ref[...] = jnp.dot(z, v, preferred_element_type=F32)


def _tc_mid(acc1, rden1, zlin, c1b, wsrc2, asrc2, wdst2, adst2):
    return pl.pallas_call(
        _mid_body,
        out_shape=[
            jax.ShapeDtypeStruct((N, D), F32),
            jax.ShapeDtypeStruct((N, 1), F32),
            jax.ShapeDtypeStruct((N, 1), F32),
        ],
    )(acc1, rden1, zlin, c1b, wsrc2, asrc2, wdst2, adst2)


def _post_body(acc_ref, rden_ref, c2b_ref, out_ref):
    s = acc_ref[0] + acc_ref[1]
    out_ref[...] = s[:, :L] * rden_ref[...] + c2b_ref[...]


def _tc_post(acc2, rden2, c2b):
    return pl.pallas_call(
        _post_body,
        out_shape=jax.ShapeDtypeStruct((N, L), F32),
    )(acc2, rden2, c2b)


_MESH = plsc.VectorSubcoreMesh(core_axis_name="c", subcore_axis_name="s")

_SC_CP = pltpu.CompilerParams()
if "needs_layout_passes" in pltpu.CompilerParams.__dataclass_fields__:
    _SC_CP = dataclasses.replace(_SC_CP, needs_layout_passes=False)


def _lrelu_exp(raw):
    return jnp.exp(jnp.where(raw >= 0.0, raw, raw * F32(0.2)))


@functools.partial(
    pl.kernel,
    out_type=jax.ShapeDtypeStruct((NW, N), F32),
    mesh=_MESH,
    scratch_types=[
        pltpu.VMEM((N,), F32),
        pltpu.VMEM((N,), F32),
        pltpu.VMEM((N,), F32),
        pltpu.VMEM((EPW,), jnp.int32),
        pltpu.VMEM((EPW,), jnp.int32),
    ],
    compiler_params=_SC_CP,
)
def _sc_denom(src_hbm, dst_hbm, asrc_hbm, adst_hbm, out_hbm,
              asrc_v, adst_v, den_v, src_v, dst_v):
    cid = lax.axis_index("c")
    sid = lax.axis_index("s")
    wid = cid * NS + sid

    pltpu.sync_copy(asrc_hbm, asrc_v)
    pltpu.sync_copy(adst_hbm, adst_v)
    pltpu.sync_copy(src_hbm.at[pl.ds(wid * EPW, EPW)], src_v)
    pltpu.sync_copy(dst_hbm.at[pl.ds(wid * EPW, EPW)], dst_v)

    z16 = jnp.zeros((16,), F32)

    @pl.loop(0, N, step=16)
    def _(i):
        den_v[pl.ds(i, 16)] = z16

    @pl.loop(0, EPW, step=16)
    def _(i):
        s16 = src_v[pl.ds(i, 16)]
        d16 = dst_v[pl.ds(i, 16)]
        raw = plsc.load_gather(asrc_v, [s16]) + plsc.load_gather(adst_v, [d16])
        plsc.addupdate_scatter(den_v, [d16], _lrelu_exp(raw))

    pltpu.sync_copy(den_v, out_hbm.at[wid])


@functools.partial(
    pl.kernel,
    out_type=jax.ShapeDtypeStruct((NC, N, D), F32),
    mesh=_MESH,
    scratch_types=[
        pltpu.VMEM((N,), F32),
        pltpu.VMEM((N,), F32),
        pltpu.VMEM((2, K), jnp.int32),
        pltpu.VMEM((2, K), jnp.int32),
        pltpu.VMEM((2, K, D), F32),
        pltpu.VMEM((K,), F32),
        pltpu.VMEM_SHARED((N, D), F32),
        pltpu.SemaphoreType.DMA,
        pltpu.SemaphoreType.DMA,
    ],
    compiler_params=_SC_CP,
)
def _sc_messages(h_hbm, src_hbm, dst_hbm, asrc_hbm, adst_hbm, out_hbm,
                 asrc_v, adst_v, src_v, dst_v, rows_v, w_v,
                 acc_sh, sg0, sg1):
    cid = lax.axis_index("c")
    sid = lax.axis_index("s")
    wid = cid * NS + sid
    ebase = wid * EPW
    sem_g = (sg0, sg1)

    pltpu.sync_copy(asrc_hbm, asrc_v)
    pltpu.sync_copy(adst_hbm, adst_v)

    z16 = jnp.zeros((16,), F32)

    @pl.loop(0, K)
    def _(r):
        @pl.loop(0, D, step=16)
        def _(j):
            rows_v[0, r, pl.ds(j, 16)] = z16

    @pl.loop(sid, NRB, step=NS)
    def _(m):
        pltpu.sync_copy(rows_v.at[0], acc_sh.at[pl.ds(m * RB, RB)])

    plsc.subcore_barrier()

    def fetch(c, b):
        off = ebase + c * K
        pltpu.sync_copy(src_hbm.at[pl.ds(off, K)], src_v.at[b])
        pltpu.sync_copy(dst_hbm.at[pl.ds(off, K)], dst_v.at[b])
        pltpu.async_copy(h_hbm.at[src_v.at[b]], rows_v.at[b], sem_g[b])

    def process(b):
        @pl.loop(0, K, step=16)
        def _(i):
            s16 = src_v[b, pl.ds(i, 16)]
            d16 = dst_v[b, pl.ds(i, 16)]
            raw = (plsc.load_gather(asrc_v, [s16])
                   + plsc.load_gather(adst_v, [d16]))
            w_v[pl.ds(i, 16)] = _lrelu_exp(raw)

        pltpu.make_async_copy(h_hbm.at[src_v.at[b]], rows_v.at[b],
                              sem_g[b]).wait()

        @pl.loop(0, K)
        def _(r):
            ws = plsc.load_gather(w_v, [jnp.full((16,), r, jnp.int32)])
            for j in range(0, D, 16):
                rows_v[b, r, pl.ds(j, 16)] = rows_v[b, r, pl.ds(j, 16)] * ws

        pltpu.sync_copy(rows_v.at[b], acc_sh.at[dst_v.at[b]], add=True)

    fetch(0, 0)

    @pl.loop(0, (NCHUNK - 1) // 2)
    def _(g):
        for b in range(2):
            c = 2 * g + b
            fetch(c + 1, 1 - b)
            process(b)

    process(0)

    plsc.subcore_barrier()

    @pl.loop(sid, NRB, step=NS)
    def _(m):
        pltpu.sync_copy(acc_sh.at[pl.ds(m * RB, RB)],
                        out_hbm.at[cid, pl.ds(m * RB, RB)])


def kernel(x, edge_index, lin_W, lin_b, c1_Wsrc, c1_Wdst, c1_asrc, c1_adst,
           c1_b, c2_Wsrc, c2_Wdst, c2_asrc, c2_adst, c2_b):
    src = edge_index[0]
    dst = edge_index[1]

    h1, as1, ad1, zlin = _tc_pre1(
        x, c1_Wsrc, c1_asrc.reshape(1, H), c1_Wdst, c1_adst.reshape(1, H),
        lin_W, lin_b.reshape(1, H))
    dp1 = _sc_denom(src, dst, as1.reshape(N), ad1.reshape(N))
    acc1 = _sc_messages(h1, src, dst, as1.reshape(N), ad1.reshape(N))
    rden1 = _tc_rcp(dp1).reshape(N, 1)

    h2, as2, ad2 = _tc_mid(
        acc1, rden1, zlin, c1_b.reshape(1, H), c2_Wsrc, c2_asrc.reshape(1, L),
        c2_Wdst, c2_adst.reshape(1, L))
    dp2 = _sc_denom(src, dst, as2.reshape(N), ad2.reshape(N))
    acc2 = _sc_messages(h2, src, dst, as2.reshape(N), ad2.reshape(N))
    rden2 = _tc_rcp(dp2).reshape(N, 1)

    return _tc_post(acc2, rden2, c2_b.reshape(1, L))


# speedup vs baseline: 39.0878x; 1.3446x over previous
import dataclasses
import functools

import jax
import jax.numpy as jnp
from jax import lax
from jax.experimental import pallas as pl
from jax.experimental.pallas import tpu as pltpu
from jax.experimental.pallas import tpu_sc as plsc

N = 10000
E = 320000
D = 128
H = 128
L = 64

NC = 2
NS = 16
NW = NC * NS
EPW = E // NW
K = 80
NCHUNK = EPW // K
RB = 80
NRB = N // RB

F32 = jnp.float32


def _pre1_body(x_ref, wsrc_ref, asrc_ref, wdst_ref, adst_ref, linw_ref,
               linb_ref, h_ref, as_ref, ad_ref, zlin_ref):
    x = x_ref[...]
    h = jnp.dot(x, wsrc_ref[...], preferred_element_type=F32)
    h_ref[...] = h
    as_ref[...] = lax.dot_general(h, asrc_ref[...],
                                  (((1,), (1,)), ((), ())),
                                  preferred_element_type=F32)
    v = lax.dot_general(wdst_ref[...], adst_ref[...],
                        (((1,), (1,)), ((), ())),
                        preferred_element_type=F32)
    ad_ref[...] = jnp.dot(x, v, preferred_element_type=F32)
    zlin_ref[...] = lax.dot_general(x, linw_ref[...],
                                    (((1,), (1,)), ((), ())),
                                    preferred_element_type=F32) + linb_ref[...]


def _tc_pre1(x, wsrc, asrc, wdst, adst, linw, linb):
    return pl.pallas_call(
        _pre1_body,
        out_shape=[
            jax.ShapeDtypeStruct((N, H), F32),
            jax.ShapeDtypeStruct((N, 1), F32),
            jax.ShapeDtypeStruct((N, 1), F32),
            jax.ShapeDtypeStruct((N, D), F32),
        ],
    )(x, wsrc, asrc, wdst, adst, linw, linb)


def _rcp_body(dp_ref, out_ref):
    out_ref[...] = 1.0 / (jnp.sum(dp_ref[...], axis=0, keepdims=True) + 1e-16)


def _tc_rcp(den_parts):
    return pl.pallas_call(
        _rcp_body,
        out_shape=jax.ShapeDtypeStruct((1, N), F32),
    )(den_parts)


def _mid_body(acc_ref, rden_ref, zlin_ref, c1b_ref, wsrc_ref, asrc_ref,
              wdst_ref, adst_ref, h_ref, as_ref, ad_ref):
    g = (acc_ref[0] + acc_ref[1]) * rden_ref[...] + c1b_ref[...]
    z = jnp.maximum(g + zlin_ref[...], 0.0)
    h = jnp.dot(z, wsrc_ref[...], preferred_element_type=F32)
    h_ref[...] = jnp.concatenate([h, jnp.zeros((N, D - L), F32)], axis=1)
    as_ref[...] = lax.dot_general(h, asrc_ref[...],
                                  (((1,), (1,)), ((), ())),
                                  preferred_element_type=F32)
    v = lax.dot_general(wdst_ref[...], adst_ref[...],
                        (((1,), (1,)), ((), ())),
                        preferred_element_type=F32)
    ad_ref[...] = jnp.dot(z, v, preferred_element_type=F32)


def _tc_mid(acc1, rden1, zlin, c1b, wsrc2, asrc2, wdst2, adst2):
    return pl.pallas_call(
        _mid_body,
        out_shape=[
            jax.ShapeDtypeStruct((N, D), F32),
            jax.ShapeDtypeStruct((N, 1), F32),
            jax.ShapeDtypeStruct((N, 1), F32),
        ],
    )(acc1, rden1, zlin, c1b, wsrc2, asrc2, wdst2, adst2)


def _post_body(acc_ref, rden_ref, c2b_ref, out_ref):
    s = acc_ref[0] + acc_ref[1]
    out_ref[...] = s[:, :L] * rden_ref[...] + c2b_ref[...]


def _tc_post(acc2, rden2, c2b):
    return pl.pallas_call(
        _post_body,
        out_shape=jax.ShapeDtypeStruct((N, L), F32),
    )(acc2, rden2, c2b)


_MESH = plsc.VectorSubcoreMesh(core_axis_name="c", subcore_axis_name="s")

_SC_CP = pltpu.CompilerParams()
if "needs_layout_passes" in pltpu.CompilerParams.__dataclass_fields__:
    _SC_CP = dataclasses.replace(_SC_CP, needs_layout_passes=False)


def _lrelu_exp(raw):
    return jnp.exp(jnp.where(raw >= 0.0, raw, raw * F32(0.2)))


@functools.partial(
    pl.kernel,
    out_type=jax.ShapeDtypeStruct((NW, N), F32),
    mesh=_MESH,
    scratch_types=[
        pltpu.VMEM((N,), F32),
        pltpu.VMEM((N,), F32),
        pltpu.VMEM((N,), F32),
        pltpu.VMEM((EPW,), jnp.int32),
        pltpu.VMEM((EPW,), jnp.int32),
    ],
    compiler_params=_SC_CP,
)
def _sc_denom(src_hbm, dst_hbm, asrc_hbm, adst_hbm, out_hbm,
              asrc_v, adst_v, den_v, src_v, dst_v):
    cid = lax.axis_index("c")
    sid = lax.axis_index("s")
    wid = cid * NS + sid

    pltpu.sync_copy(asrc_hbm, asrc_v)
    pltpu.sync_copy(adst_hbm, adst_v)
    pltpu.sync_copy(src_hbm.at[pl.ds(wid * EPW, EPW)], src_v)
    pltpu.sync_copy(dst_hbm.at[pl.ds(wid * EPW, EPW)], dst_v)

    z16 = jnp.zeros((16,), F32)

    @pl.loop(0, N, step=16)
    def _(i):
        den_v[pl.ds(i, 16)] = z16

    @pl.loop(0, EPW, step=16)
    def _(i):
        s16 = src_v[pl.ds(i, 16)]
        d16 = dst_v[pl.ds(i, 16)]
        raw = plsc.load_gather(asrc_v, [s16]) + plsc.load_gather(adst_v, [d16])
        plsc.addupdate_scatter(den_v, [d16], _lrelu_exp(raw))

    pltpu.sync_copy(den_v, out_hbm.at[wid])


@functools.partial(
    pl.kernel,
    out_type=jax.ShapeDtypeStruct((NC, N, D), F32),
    mesh=_MESH,
    scratch_types=[
        pltpu.VMEM((N,), F32),
        pltpu.VMEM((N,), F32),
        pltpu.VMEM((2, 2, K), jnp.int32),
        pltpu.VMEM((2, K, D), F32),
        pltpu.VMEM((K,), F32),
        pltpu.VMEM_SHARED((N, D), F32),
        pltpu.SemaphoreType.DMA,
        pltpu.SemaphoreType.DMA,
        pltpu.SemaphoreType.DMA,
        pltpu.SemaphoreType.DMA,
    ],
    compiler_params=_SC_CP,
)
def _sc_messages(h_hbm, ei_hbm, asrc_hbm, adst_hbm, out_hbm,
                 asrc_v, adst_v, idx_v, rows_v, w_v,
                 acc_sh, sg0, sg1, si0, si1):
    cid = lax.axis_index("c")
    sid = lax.axis_index("s")
    wid = cid * NS + sid
    cbase = wid * NCHUNK
    sem_g = (sg0, sg1)
    sem_i = (si0, si1)

    pltpu.sync_copy(asrc_hbm, asrc_v)
    pltpu.sync_copy(adst_hbm, adst_v)

    z16 = jnp.zeros((16,), F32)

    @pl.loop(0, K)
    def _(r):
        @pl.loop(0, D, step=16)
        def _(j):
            rows_v[0, r, pl.ds(j, 16)] = z16

    @pl.loop(sid, NRB, step=NS)
    def _(m):
        pltpu.sync_copy(rows_v.at[0], acc_sh.at[pl.ds(m * RB, RB)])

    plsc.subcore_barrier()

    def fetch_idx(c, b):
        pltpu.async_copy(ei_hbm.at[cbase + c], idx_v.at[b], sem_i[b])

    def wait_idx(b):
        pltpu.make_async_copy(ei_hbm.at[cbase], idx_v.at[b], sem_i[b]).wait()

    def start_gather(b):
        pltpu.async_copy(h_hbm.at[idx_v.at[b, 0]], rows_v.at[b], sem_g[b])

    def process(c, b):
        @pl.loop(0, K, step=16)
        def _(i):
            s16 = idx_v[b, 0, pl.ds(i, 16)]
            d16 = idx_v[b, 1, pl.ds(i, 16)]
            raw = (plsc.load_gather(asrc_v, [s16])
                   + plsc.load_gather(adst_v, [d16]))
            w_v[pl.ds(i, 16)] = _lrelu_exp(raw)

        pltpu.make_async_copy(h_hbm.at[idx_v.at[b, 0]], rows_v.at[b],
                              sem_g[b]).wait()

        @plsc.parallel_loop(0, K, unroll=2)
        def _(r):
            ws = plsc.load_gather(w_v, [jnp.full((16,), r, jnp.int32)])
            for j in range(0, D, 16):
                rows_v[b, r, pl.ds(j, 16)] = rows_v[b, r, pl.ds(j, 16)] * ws

        pltpu.sync_copy(rows_v.at[b], acc_sh.at[idx_v.at[b, 1]], add=True)
        fetch_idx(jnp.minimum(c + 2, NCHUNK - 1), b)

    fetch_idx(0, 0)
    fetch_idx(1, 1)
    wait_idx(0)
    start_gather(0)
    wait_idx(1)
    process(0, 0)
    start_gather(1)
    wait_idx(0)
    process(1, 1)
    start_gather(0)

    @pl.loop(1, (NCHUNK - 1) // 2)
    def _(g):
        for b in range(2):
            c = 2 * g + b
            wait_idx(1 - b)
            start_gather(1 - b)
            process(c, b)

    wait_idx(1)
    process(NCHUNK - 1, 0)
    wait_idx(0)

    plsc.subcore_barrier()

    @pl.loop(sid, NRB, step=NS)
    def _(m):
        pltpu.sync_copy(acc_sh.at[pl.ds(m * RB, RB)],
                        out_hbm.at[cid, pl.ds(m * RB, RB)])


def kernel(x, edge_index, lin_W, lin_b, c1_Wsrc, c1_Wdst, c1_asrc, c1_adst,
           c1_b, c2_Wsrc, c2_Wdst, c2_asrc, c2_adst, c2_b):
    src = edge_index[0]
    dst = edge_index[1]
    ei3 = edge_index.reshape(2, NW * NCHUNK, K).transpose(1, 0, 2)

    h1, as1, ad1, zlin = _tc_pre1(
        x, c1_Wsrc, c1_asrc.reshape(1, H), c1_Wdst, c1_adst.reshape(1, H),
        lin_W, lin_b.reshape(1, H))
    dp1 = _sc_denom(src, dst, as1.reshape(N), ad1.reshape(N))
    acc1 = _sc_messages(h1, ei3, as1.reshape(N), ad1.reshape(N))
    rden1 = _tc_rcp(dp1).reshape(N, 1)

    h2, as2, ad2 = _tc_mid(
        acc1, rden1, zlin, c1_b.reshape(1, H), c2_Wsrc, c2_asrc.reshape(1, L),
        c2_Wdst, c2_adst.reshape(1, L))
    dp2 = _sc_denom(src, dst, as2.reshape(N), ad2.reshape(N))
    acc2 = _sc_messages(h2, ei3, as2.reshape(N), ad2.reshape(N))
    rden2 = _tc_rcp(dp2).reshape(N, 1)

    return _tc_post(acc2, rden2, c2_b.reshape(1, L))


# speedup vs baseline: 54.7799x; 1.4015x over previous
import dataclasses
import functools

import jax
import jax.numpy as jnp
from jax import lax
from jax.experimental import pallas as pl
from jax.experimental.pallas import tpu as pltpu
from jax.experimental.pallas import tpu_sc as plsc

N = 10000
E = 320000
D = 128
H = 128
L = 64

NC = 2
NS = 16
NW = NC * NS
EPW = E // NW
K = 80
NCHUNK = EPW // K
RB = 80
NRB = N // RB

F32 = jnp.float32


def _pre1_body(x_ref, wsrc_ref, asrc_ref, wdst_ref, adst_ref, linw_ref,
               linb_ref, h_ref, as_ref, ad_ref, zlin_ref):
    x = x_ref[...]
    h = jnp.dot(x, wsrc_ref[...], preferred_element_type=F32)
    h_ref[...] = h
    as_ref[...] = lax.dot_general(h, asrc_ref[...],
                                  (((1,), (1,)), ((), ())),
                                  preferred_element_type=F32)
    v = lax.dot_general(wdst_ref[...], adst_ref[...],
                        (((1,), (1,)), ((), ())),
                        preferred_element_type=F32)
    ad_ref[...] = jnp.dot(x, v, preferred_element_type=F32)
    zlin_ref[...] = lax.dot_general(x, linw_ref[...],
                                    (((1,), (1,)), ((), ())),
                                    preferred_element_type=F32) + linb_ref[...]


def _tc_pre1(x, wsrc, asrc, wdst, adst, linw, linb):
    return pl.pallas_call(
        _pre1_body,
        out_shape=[
            jax.ShapeDtypeStruct((N, H), F32),
            jax.ShapeDtypeStruct((N, 1), F32),
            jax.ShapeDtypeStruct((N, 1), F32),
            jax.ShapeDtypeStruct((N, D), F32),
        ],
    )(x, wsrc, asrc, wdst, adst, linw, linb)


def _rcp_body(dp_ref, out_ref):
    out_ref[...] = 1.0 / (jnp.sum(dp_ref[...], axis=0, keepdims=True) + 1e-16)


def _tc_rcp(den_parts):
    return pl.pallas_call(
        _rcp_body,
        out_shape=jax.ShapeDtypeStruct((1, N), F32),
    )(den_parts)


def _mid_body(acc_ref, rden_ref, zlin_ref, c1b_ref, wsrc_ref, asrc_ref,
              wdst_ref, adst_ref, h_ref, as_ref, ad_ref):
    g = (acc_ref[0] + acc_ref[1]) * rden_ref[...] + c1b_ref[...]
    z = jnp.maximum(g + zlin_ref[...], 0.0)
    h = jnp.dot(z, wsrc_ref[...], preferred_element_type=F32)
    h_ref[...] = jnp.concatenate([h, jnp.zeros((N, D - L), F32)], axis=1)
    as_ref[...] = lax.dot_general(h, asrc_ref[...],
                                  (((1,), (1,)), ((), ())),
                                  preferred_element_type=F32)
    v = lax.dot_general(wdst_ref[...], adst_ref[...],
                        (((1,), (1,)), ((), ())),
                        preferred_element_type=F32)
    ad_ref[...] = jnp.dot(z, v, preferred_element_type=F32)


def _tc_mid(acc1, rden1, zlin, c1b, wsrc2, asrc2, wdst2, adst2):
    return pl.pallas_call(
        _mid_body,
        out_shape=[
            jax.ShapeDtypeStruct((N, D), F32),
            jax.ShapeDtypeStruct((N, 1), F32),
            jax.ShapeDtypeStruct((N, 1), F32),
        ],
    )(acc1, rden1, zlin, c1b, wsrc2, asrc2, wdst2, adst2)


def _post_body(acc_ref, rden_ref, c2b_ref, out_ref):
    s = acc_ref[0] + acc_ref[1]
    out_ref[...] = s[:, :L] * rden_ref[...] + c2b_ref[...]


def _tc_post(acc2, rden2, c2b):
    return pl.pallas_call(
        _post_body,
        out_shape=jax.ShapeDtypeStruct((N, L), F32),
    )(acc2, rden2, c2b)


_MESH = plsc.VectorSubcoreMesh(core_axis_name="c", subcore_axis_name="s")

_SC_CP = pltpu.CompilerParams()
if "needs_layout_passes" in pltpu.CompilerParams.__dataclass_fields__:
    _SC_CP = dataclasses.replace(_SC_CP, needs_layout_passes=False)


def _lrelu_exp(raw):
    return jnp.exp(jnp.where(raw >= 0.0, raw, raw * F32(0.2)))


WSUP = 2000
NSUP = EPW // WSUP


@functools.partial(
    pl.kernel,
    out_type=[
        jax.ShapeDtypeStruct((NW, N), F32),
        jax.ShapeDtypeStruct((E,), F32),
    ],
    mesh=_MESH,
    scratch_types=[
        pltpu.VMEM((N,), F32),
        pltpu.VMEM((N,), F32),
        pltpu.VMEM((N,), F32),
        pltpu.VMEM((EPW,), jnp.int32),
        pltpu.VMEM((EPW,), jnp.int32),
        pltpu.VMEM((WSUP,), F32),
    ],
    compiler_params=_SC_CP,
)
def _sc_denom(src_hbm, dst_hbm, asrc_hbm, adst_hbm, out_hbm, w_hbm,
              asrc_v, adst_v, den_v, src_v, dst_v, w_buf):
    cid = lax.axis_index("c")
    sid = lax.axis_index("s")
    wid = cid * NS + sid

    pltpu.sync_copy(asrc_hbm, asrc_v)
    pltpu.sync_copy(adst_hbm, adst_v)
    pltpu.sync_copy(src_hbm.at[pl.ds(wid * EPW, EPW)], src_v)
    pltpu.sync_copy(dst_hbm.at[pl.ds(wid * EPW, EPW)], dst_v)

    z16 = jnp.zeros((16,), F32)

    @pl.loop(0, N, step=16)
    def _(i):
        den_v[pl.ds(i, 16)] = z16

    @pl.loop(0, NSUP)
    def _(t):
        @pl.loop(0, WSUP, step=16)
        def _(i):
            e = t * WSUP + i
            s16 = src_v[pl.ds(e, 16)]
            d16 = dst_v[pl.ds(e, 16)]
            raw = (plsc.load_gather(asrc_v, [s16])
                   + plsc.load_gather(adst_v, [d16]))
            w = _lrelu_exp(raw)
            plsc.addupdate_scatter(den_v, [d16], w)
            w_buf[pl.ds(i, 16)] = w

        pltpu.sync_copy(w_buf, w_hbm.at[pl.ds(wid * EPW + t * WSUP, WSUP)])

    pltpu.sync_copy(den_v, out_hbm.at[wid])


@functools.partial(
    pl.kernel,
    out_type=jax.ShapeDtypeStruct((NC, N, D), F32),
    mesh=_MESH,
    scratch_types=[
        pltpu.VMEM((3, 2, K), jnp.int32),
        pltpu.VMEM((3, K), jnp.int32),
        pltpu.VMEM((3, K), F32),
        pltpu.VMEM((3, K, D), F32),
        pltpu.VMEM_SHARED((N, D), F32),
        pltpu.SemaphoreType.DMA,
        pltpu.SemaphoreType.DMA,
        pltpu.SemaphoreType.DMA,
        pltpu.SemaphoreType.DMA,
        pltpu.SemaphoreType.DMA,
        pltpu.SemaphoreType.DMA,
        pltpu.SemaphoreType.DMA,
        pltpu.SemaphoreType.DMA,
        pltpu.SemaphoreType.DMA,
    ],
    compiler_params=_SC_CP,
)
def _sc_messages(h_hbm, ei_hbm, w_hbm, out_hbm,
                 idx_v, sidx_v, w_v, rows_v,
                 acc_sh, si0, si1, si2, sg0, sg1, sg2, ss0, ss1, ss2):
    cid = lax.axis_index("c")
    sid = lax.axis_index("s")
    wid = cid * NS + sid
    cbase = wid * NCHUNK
    wbase = wid * EPW
    sem_i = (si0, si1, si2)
    sem_g = (sg0, sg1, sg2)
    sem_s = (ss0, ss1, ss2)

    z16 = jnp.zeros((16,), F32)

    @pl.loop(0, K)
    def _(r):
        @pl.loop(0, D, step=16)
        def _(j):
            rows_v[0, r, pl.ds(j, 16)] = z16

    @pl.loop(sid, NRB, step=NS)
    def _(m):
        pltpu.sync_copy(rows_v.at[0], acc_sh.at[pl.ds(m * RB, RB)])

    plsc.subcore_barrier()

    def fetch_idxw(c, b):
        pltpu.async_copy(ei_hbm.at[cbase + c], idx_v.at[b], sem_i[b])
        pltpu.async_copy(w_hbm.at[pl.ds(wbase + c * K, K)], w_v.at[b],
                         sem_i[b])

    def wait_idxw(b):
        pltpu.make_async_copy(ei_hbm.at[cbase], idx_v.at[b], sem_i[b]).wait()
        pltpu.make_async_copy(w_hbm.at[pl.ds(wbase, K)], w_v.at[b],
                              sem_i[b]).wait()

    def start_gather(b):
        pltpu.async_copy(h_hbm.at[idx_v.at[b, 0]], rows_v.at[b], sem_g[b])

    def wait_scatter(b):
        pltpu.make_async_copy(rows_v.at[b], acc_sh.at[sidx_v.at[b]],
                              sem_s[b]).wait()

    def process(b):
        pltpu.make_async_copy(h_hbm.at[idx_v.at[b, 0]], rows_v.at[b],
                              sem_g[b]).wait()

        @plsc.parallel_loop(0, K, unroll=2)
        def _(r):
            ws = plsc.load_gather(w_v.at[b], [jnp.full((16,), r, jnp.int32)])
            for j in range(0, D, 16):
                rows_v[b, r, pl.ds(j, 16)] = rows_v[b, r, pl.ds(j, 16)] * ws

        @pl.loop(0, K, step=16)
        def _(i):
            sidx_v[b, pl.ds(i, 16)] = idx_v[b, 1, pl.ds(i, 16)]

        pltpu.async_copy(rows_v.at[b], acc_sh.at[sidx_v.at[b]], sem_s[b],
                         add=True)

    def body(c, b, first):
        bn = (b + 1) % 3
        bp = (b + 2) % 3
        wait_idxw(bn)
        if not first:
            wait_scatter(bn)
        start_gather(bn)
        fetch_idxw(jnp.minimum(c + 2, NCHUNK - 1), bp)
        process(b)

    fetch_idxw(0, 0)
    fetch_idxw(1, 1)
    wait_idxw(0)
    start_gather(0)
    body(0, 0, True)
    body(1, 1, True)

    @pl.loop(0, (NCHUNK - 2) // 3)
    def _(g):
        for k in range(3):
            c = 2 + 3 * g + k
            body(c, (2 + k) % 3, False)

    pltpu.make_async_copy(h_hbm.at[idx_v.at[2, 0]], rows_v.at[2],
                          sem_g[2]).wait()
    wait_scatter(0)
    wait_scatter(1)
    wait_idxw(0)

    plsc.subcore_barrier()

    @pl.loop(sid, NRB, step=NS)
    def _(m):
        pltpu.sync_copy(acc_sh.at[pl.ds(m * RB, RB)],
                        out_hbm.at[cid, pl.ds(m * RB, RB)])


def kernel(x, edge_index, lin_W, lin_b, c1_Wsrc, c1_Wdst, c1_asrc, c1_adst,
           c1_b, c2_Wsrc, c2_Wdst, c2_asrc, c2_adst, c2_b):
    src = edge_index[0]
    dst = edge_index[1]
    ei3 = edge_index.reshape(2, NW * NCHUNK, K).transpose(1, 0, 2)

    h1, as1, ad1, zlin = _tc_pre1(
        x, c1_Wsrc, c1_asrc.reshape(1, H), c1_Wdst, c1_adst.reshape(1, H),
        lin_W, lin_b.reshape(1, H))
    dp1, w1 = _sc_denom(src, dst, as1.reshape(N), ad1.reshape(N))
    acc1 = _sc_messages(h1, ei3, w1)
    rden1 = _tc_rcp(dp1).reshape(N, 1)

    h2, as2, ad2 = _tc_mid(
        acc1, rden1, zlin, c1_b.reshape(1, H), c2_Wsrc, c2_asrc.reshape(1, L),
        c2_Wdst, c2_adst.reshape(1, L))
    dp2, w2 = _sc_denom(src, dst, as2.reshape(N), ad2.reshape(N))
    acc2 = _sc_messages(h2, ei3, w2)
    rden2 = _tc_rcp(dp2).reshape(N, 1)

    return _tc_post(acc2, rden2, c2_b.reshape(1, L))


# speedup vs baseline: 55.7286x; 1.0173x over previous
import dataclasses
import functools

import jax
import jax.numpy as jnp
from jax import lax
from jax.experimental import pallas as pl
from jax.experimental.pallas import tpu as pltpu
from jax.experimental.pallas import tpu_sc as plsc

N = 10000
E = 320000
D = 128
H = 128
L = 64

NC = 2
NS = 16
NW = NC * NS
EPW = E // NW
K = 80
NCHUNK = EPW // K
RB = 80
NRB = N // RB

F32 = jnp.float32


def _pre1_body(x_ref, wsrc_ref, asrc_ref, wdst_ref, adst_ref, linw_ref,
               linb_ref, h_ref, as_ref, ad_ref, zlin_ref):
    x = x_ref[...]
    h = jnp.dot(x, wsrc_ref[...], preferred_element_type=F32)
    h_ref[...] = h
    as_ref[...] = lax.dot_general(h, asrc_ref[...],
                                  (((1,), (1,)), ((), ())),
                                  preferred_element_type=F32)
    v = lax.dot_general(wdst_ref[...], adst_ref[...],
                        (((1,), (1,)), ((), ())),
                        preferred_element_type=F32)
    ad_ref[...] = jnp.dot(x, v, preferred_element_type=F32)
    zlin_ref[...] = lax.dot_general(x, linw_ref[...],
                                    (((1,), (1,)), ((), ())),
                                    preferred_element_type=F32) + linb_ref[...]


def _tc_pre1(x, wsrc, asrc, wdst, adst, linw, linb):
    return pl.pallas_call(
        _pre1_body,
        out_shape=[
            jax.ShapeDtypeStruct((N, H), F32),
            jax.ShapeDtypeStruct((N, 1), F32),
            jax.ShapeDtypeStruct((N, 1), F32),
            jax.ShapeDtypeStruct((N, D), F32),
        ],
    )(x, wsrc, asrc, wdst, adst, linw, linb)


def _rcp_body(dp_ref, out_ref):
    out_ref[...] = 1.0 / (jnp.sum(dp_ref[...], axis=0, keepdims=True) + 1e-16)


def _tc_rcp(den_parts):
    return pl.pallas_call(
        _rcp_body,
        out_shape=jax.ShapeDtypeStruct((1, N), F32),
    )(den_parts)


def _mid_body(acc_ref, rden_ref, zlin_ref, c1b_ref, wsrc_ref, asrc_ref,
              wdst_ref, adst_ref, h_ref, as_ref, ad_ref):
    g = (acc_ref[0] + acc_ref[1]) * rden_ref[...] + c1b_ref[...]
    z = jnp.maximum(g + zlin_ref[...], 0.0)
    h = jnp.dot(z, wsrc_ref[...], preferred_element_type=F32)
    h_ref[...] = jnp.concatenate([h, jnp.zeros((N, D - L), F32)], axis=1)
    as_ref[...] = lax.dot_general(h, asrc_ref[...],
                                  (((1,), (1,)), ((), ())),
                                  preferred_element_type=F32)
    v = lax.dot_general(wdst_ref[...], adst_ref[...],
                        (((1,), (1,)), ((), ())),
                        preferred_element_type=F32)
    ad_ref[...] = jnp.dot(z, v, preferred_element_type=F32)


def _tc_mid(acc1, rden1, zlin, c1b, wsrc2, asrc2, wdst2, adst2):
    return pl.pallas_call(
        _mid_body,
        out_shape=[
            jax.ShapeDtypeStruct((N, D), F32),
            jax.ShapeDtypeStruct((N, 1), F32),
            jax.ShapeDtypeStruct((N, 1), F32),
        ],
    )(acc1, rden1, zlin, c1b, wsrc2, asrc2, wdst2, adst2)


def _post_body(acc_ref, rden_ref, c2b_ref, out_ref):
    s = acc_ref[0] + acc_ref[1]
    out_ref[...] = s[:, :L] * rden_ref[...] + c2b_ref[...]


def _tc_post(acc2, rden2, c2b):
    return pl.pallas_call(
        _post_body,
        out_shape=jax.ShapeDtypeStruct((N, L), F32),
    )(acc2, rden2, c2b)


_MESH = plsc.VectorSubcoreMesh(core_axis_name="c", subcore_axis_name="s")

_SC_CP = pltpu.CompilerParams()
if "needs_layout_passes" in pltpu.CompilerParams.__dataclass_fields__:
    _SC_CP = dataclasses.replace(_SC_CP, needs_layout_passes=False)


def _lrelu_exp(raw):
    return jnp.exp(jnp.where(raw >= 0.0, raw, raw * F32(0.2)))


WSUP = 2000
NSUP = EPW // WSUP


@functools.partial(
    pl.kernel,
    out_type=[
        jax.ShapeDtypeStruct((NW, N), F32),
        jax.ShapeDtypeStruct((E,), F32),
    ],
    mesh=_MESH,
    scratch_types=[
        pltpu.VMEM((N,), F32),
        pltpu.VMEM((N,), F32),
        pltpu.VMEM((N,), F32),
        pltpu.VMEM((EPW,), jnp.int32),
        pltpu.VMEM((EPW,), jnp.int32),
        pltpu.VMEM((WSUP,), F32),
        pltpu.SemaphoreType.DMA,
    ],
    compiler_params=_SC_CP,
)
def _sc_denom(src_hbm, dst_hbm, asrc_hbm, adst_hbm, out_hbm, w_hbm,
              asrc_v, adst_v, den_v, src_v, dst_v, w_buf, sem):
    cid = lax.axis_index("c")
    sid = lax.axis_index("s")
    wid = cid * NS + sid

    cps = [
        pltpu.async_copy(asrc_hbm, asrc_v, sem),
        pltpu.async_copy(adst_hbm, adst_v, sem),
        pltpu.async_copy(src_hbm.at[pl.ds(wid * EPW, EPW)], src_v, sem),
        pltpu.async_copy(dst_hbm.at[pl.ds(wid * EPW, EPW)], dst_v, sem),
    ]

    z16 = jnp.zeros((16,), F32)

    @pl.loop(0, N, step=16)
    def _(i):
        den_v[pl.ds(i, 16)] = z16

    for cp in cps:
        cp.wait()

    @pl.loop(0, NSUP)
    def _(t):
        @pl.loop(0, WSUP, step=16)
        def _(i):
            e = t * WSUP + i
            s16 = src_v[pl.ds(e, 16)]
            d16 = dst_v[pl.ds(e, 16)]
            raw = (plsc.load_gather(asrc_v, [s16])
                   + plsc.load_gather(adst_v, [d16]))
            w = _lrelu_exp(raw)
            plsc.addupdate_scatter(den_v, [d16], w)
            w_buf[pl.ds(i, 16)] = w

        pltpu.sync_copy(w_buf, w_hbm.at[pl.ds(wid * EPW + t * WSUP, WSUP)])

    pltpu.sync_copy(den_v, out_hbm.at[wid])


@functools.partial(
    pl.kernel,
    out_type=jax.ShapeDtypeStruct((NC, N, D), F32),
    mesh=_MESH,
    scratch_types=[
        pltpu.VMEM((3, 2, K), jnp.int32),
        pltpu.VMEM((3, K), jnp.int32),
        pltpu.VMEM((3, K), F32),
        pltpu.VMEM((3, K, D), F32),
        pltpu.VMEM_SHARED((N, D), F32),
        pltpu.SemaphoreType.DMA,
        pltpu.SemaphoreType.DMA,
        pltpu.SemaphoreType.DMA,
        pltpu.SemaphoreType.DMA,
        pltpu.SemaphoreType.DMA,
        pltpu.SemaphoreType.DMA,
        pltpu.SemaphoreType.DMA,
        pltpu.SemaphoreType.DMA,
        pltpu.SemaphoreType.DMA,
    ],
    compiler_params=_SC_CP,
)
def _sc_messages(h_hbm, ei_hbm, w_hbm, out_hbm,
                 idx_v, sidx_v, w_v, rows_v,
                 acc_sh, si0, si1, si2, sg0, sg1, sg2, ss0, ss1, ss2):
    cid = lax.axis_index("c")
    sid = lax.axis_index("s")
    wid = cid * NS + sid
    cbase = wid * NCHUNK
    wbase = wid * EPW
    sem_i = (si0, si1, si2)
    sem_g = (sg0, sg1, sg2)
    sem_s = (ss0, ss1, ss2)

    z16 = jnp.zeros((16,), F32)

    @pl.loop(0, K)
    def _(r):
        @pl.loop(0, D, step=16)
        def _(j):
            rows_v[0, r, pl.ds(j, 16)] = z16

    @pl.loop(sid, NRB, step=NS)
    def _(m):
        pltpu.sync_copy(rows_v.at[0], acc_sh.at[pl.ds(m * RB, RB)])

    plsc.subcore_barrier()

    def fetch_idxw(c, b):
        pltpu.async_copy(ei_hbm.at[cbase + c], idx_v.at[b], sem_i[b])
        pltpu.async_copy(w_hbm.at[pl.ds(wbase + c * K, K)], w_v.at[b],
                         sem_i[b])

    def wait_idxw(b):
        pltpu.make_async_copy(ei_hbm.at[cbase], idx_v.at[b], sem_i[b]).wait()
        pltpu.make_async_copy(w_hbm.at[pl.ds(wbase, K)], w_v.at[b],
                              sem_i[b]).wait()

    def start_gather(b):
        pltpu.async_copy(h_hbm.at[idx_v.at[b, 0]], rows_v.at[b], sem_g[b])

    def wait_scatter(b):
        pltpu.make_async_copy(rows_v.at[b], acc_sh.at[sidx_v.at[b]],
                              sem_s[b]).wait()

    def process(b):
        pltpu.make_async_copy(h_hbm.at[idx_v.at[b, 0]], rows_v.at[b],
                              sem_g[b]).wait()

        @plsc.parallel_loop(0, K, unroll=4)
        def _(r):
            ws = plsc.load_gather(w_v.at[b], [jnp.full((16,), r, jnp.int32)])
            for j in range(0, D, 16):
                rows_v[b, r, pl.ds(j, 16)] = rows_v[b, r, pl.ds(j, 16)] * ws

        @pl.loop(0, K, step=16)
        def _(i):
            sidx_v[b, pl.ds(i, 16)] = idx_v[b, 1, pl.ds(i, 16)]

        pltpu.async_copy(rows_v.at[b], acc_sh.at[sidx_v.at[b]], sem_s[b],
                         add=True)

    def body(c, b, first):
        bn = (b + 1) % 3
        bp = (b + 2) % 3
        wait_idxw(bn)
        if not first:
            wait_scatter(bn)
        start_gather(bn)
        fetch_idxw(jnp.minimum(c + 2, NCHUNK - 1), bp)
        process(b)

    fetch_idxw(0, 0)
    fetch_idxw(1, 1)
    wait_idxw(0)
    start_gather(0)
    body(0, 0, True)
    body(1, 1, True)

    @pl.loop(0, (NCHUNK - 2) // 3)
    def _(g):
        for k in range(3):
            c = 2 + 3 * g + k
            body(c, (2 + k) % 3, False)

    pltpu.make_async_copy(h_hbm.at[idx_v.at[2, 0]], rows_v.at[2],
                          sem_g[2]).wait()
    wait_scatter(0)
    wait_scatter(1)
    wait_idxw(0)

    plsc.subcore_barrier()

    @pl.loop(sid, NRB, step=NS)
    def _(m):
        pltpu.sync_copy(acc_sh.at[pl.ds(m * RB, RB)],
                        out_hbm.at[cid, pl.ds(m * RB, RB)])


def kernel(x, edge_index, lin_W, lin_b, c1_Wsrc, c1_Wdst, c1_asrc, c1_adst,
           c1_b, c2_Wsrc, c2_Wdst, c2_asrc, c2_adst, c2_b):
    src = edge_index[0]
    dst = edge_index[1]
    ei3 = edge_index.reshape(2, NW * NCHUNK, K).transpose(1, 0, 2)

    h1, as1, ad1, zlin = _tc_pre1(
        x, c1_Wsrc, c1_asrc.reshape(1, H), c1_Wdst, c1_adst.reshape(1, H),
        lin_W, lin_b.reshape(1, H))
    dp1, w1 = _sc_denom(src, dst, as1.reshape(N), ad1.reshape(N))
    acc1 = _sc_messages(h1, ei3, w1)
    rden1 = _tc_rcp(dp1).reshape(N, 1)

    h2, as2, ad2 = _tc_mid(
        acc1, rden1, zlin, c1_b.reshape(1, H), c2_Wsrc, c2_asrc.reshape(1, L),
        c2_Wdst, c2_adst.reshape(1, L))
    dp2, w2 = _sc_denom(src, dst, as2.reshape(N), ad2.reshape(N))
    acc2 = _sc_messages(h2, ei3, w2)
    rden2 = _tc_rcp(dp2).reshape(N, 1)

    return _tc_post(acc2, rden2, c2_b.reshape(1, L))
